# Initial kernel scaffold; baseline (speedup 1.0000x reference)
import jax
import jax.numpy as jnp
from jax.experimental import pallas as pl


def kernel(x, skip, px, py, pxyz, pknn, num_points, kernel_points, kp_weights, bn_gamma, bn_beta, head_w, head_b):
    raise NotImplementedError("write your pallas kernel here")


# speedup vs baseline: 2.5527x; 2.5527x over previous
import functools

import jax
import jax.numpy as jnp
from jax import lax
from jax.experimental import pallas as pl
from jax.experimental.pallas import tpu as pltpu
from jax.experimental.pallas import tpu_sc as plsc

DIM = 64
NUM_CLASSES = 17
GS_H, GS_W = 32, 128
KNN = 7
KP = 15
KP_EXTENT = 1.2

NC, NS = 2, 16
NW = NC * NS
CHUNK = 128
FW = 80


def _bcast_lane(vec, lane):
    idx = jnp.full((16, 1), lane, jnp.int32)
    return lax.gather(
        vec, idx,
        dimension_numbers=lax.GatherDimensionNumbers(
            offset_dims=(), collapsed_slice_dims=(0,), start_index_map=(0,)),
        slice_sizes=(1,),
        mode=lax.GatherScatterMode.PROMISE_IN_BOUNDS)


def _sc_mesh():
    return plsc.VectorSubcoreMesh(core_axis_name="c", subcore_axis_name="s")


def _wid():
    return lax.axis_index("s") * NC + lax.axis_index("c")


def _gs_body(n_per_w, nchunks, xt_hbm, idxa_hbm, wa_hbm, pz_hbm, out_hbm,
             idx_v, w_v, v_v, pz_v, fb, sem):
    wid = _wid()
    base = wid * n_per_w

    def chunk(c, carry):
        rows = base + c * CHUNK
        pltpu.sync_copy(idxa_hbm.at[wid, c], idx_v)
        pltpu.sync_copy(wa_hbm.at[wid, c], w_v)
        pltpu.sync_copy(pz_hbm.at[pl.ds(rows, CHUNK)], pz_v)
        descs = []
        for q in range(4):
            descs.append(pltpu.async_copy(xt_hbm.at[idx_v.at[q]], v_v.at[q], sem))
        for d in descs:
            d.wait()

        def blend(g, carry2):
            wvecs = [w_v[pl.ds(q * CHUNK + g * 16, 16)] for q in range(4)]
            for j2 in range(16):
                j = g * 16 + j2
                ws = [_bcast_lane(wvecs[q], j2) for q in range(4)]
                for cc in range(4):
                    sl = pl.ds(cc * 16, 16)
                    acc = ws[0] * v_v[0, j, sl]
                    acc = acc + ws[1] * v_v[1, j, sl]
                    acc = acc + ws[2] * v_v[2, j, sl]
                    acc = acc + ws[3] * v_v[3, j, sl]
                    fb[j, sl] = acc
                fb[j, pl.ds(64, 16)] = pz_v[j, :]
            return carry2

        lax.fori_loop(0, CHUNK // 16, blend, 0, unroll=False)
        pltpu.sync_copy(fb, out_hbm.at[pl.ds(rows, CHUNK)])
        return carry

    lax.fori_loop(0, nchunks, chunk, 0, unroll=False)


def _grid_sample_sc(xt, idxa, wa, pz16, n_total):
    n_per_w = n_total // NW
    nchunks = n_per_w // CHUNK
    body = functools.partial(_gs_body, n_per_w, nchunks)
    return pl.kernel(
        body,
        out_type=jax.ShapeDtypeStruct((n_total, FW), jnp.float32),
        mesh=_sc_mesh(),
        scratch_types=[
            pltpu.VMEM((4, CHUNK), jnp.int32),
            pltpu.VMEM((4 * CHUNK,), jnp.float32),
            pltpu.VMEM((4, CHUNK, DIM), jnp.float32),
            pltpu.VMEM((CHUNK, 16), jnp.float32),
            pltpu.VMEM((CHUNK, FW), jnp.float32),
            pltpu.SemaphoreType.DMA,
        ],
        compiler_params=pltpu.CompilerParams(use_tc_tiling_on_sc=False),
    )(xt, idxa, wa, pz16)


def _nb_body(n_per_w, nchunks, t2_hbm, idxb_hbm, out_hbm,
             idx_v, buf0, buf1, sem0, sem1):
    wid = _wid()
    base = wid * n_per_w
    pltpu.sync_copy(idxb_hbm.at[wid], idx_v)
    total = KNN * nchunks

    def issue(f, buf, sem):
        k = f // nchunks
        c = f - k * nchunks
        return pltpu.async_copy(t2_hbm.at[idx_v.at[k, c]], buf, sem)

    def wait(f, buf, sem):
        k = f // nchunks
        c = f - k * nchunks
        pltpu.make_async_copy(t2_hbm.at[idx_v.at[k, c]], buf, sem).wait()

    def store(f, buf):
        k = f // nchunks
        c = f - k * nchunks
        pltpu.sync_copy(buf, out_hbm.at[k, pl.ds(base + c * CHUNK, CHUNK)])

    issue(0, buf0, sem0)

    def pair(g, carry):
        f0 = 2 * g
        f1 = 2 * g + 1
        issue(f1, buf1, sem1)
        wait(f0, buf0, sem0)
        store(f0, buf0)

        @pl.when(f1 + 1 < total)
        def _():
            issue(f1 + 1, buf0, sem0)

        wait(f1, buf1, sem1)
        store(f1, buf1)
        return carry

    lax.fori_loop(0, total // 2, pair, 0, unroll=False)


def _neighbor_gather_sc(t2, idxb, n_total):
    n_per_w = n_total // NW
    nchunks = n_per_w // CHUNK
    body = functools.partial(_nb_body, n_per_w, nchunks)
    return pl.kernel(
        body,
        out_type=jax.ShapeDtypeStruct((KNN, n_total, FW), jnp.float32),
        mesh=_sc_mesh(),
        scratch_types=[
            pltpu.VMEM((KNN, nchunks, CHUNK), jnp.int32),
            pltpu.VMEM((CHUNK, FW), jnp.float32),
            pltpu.VMEM((CHUNK, FW), jnp.float32),
            pltpu.SemaphoreType.DMA,
            pltpu.SemaphoreType.DMA,
        ],
        compiler_params=pltpu.CompilerParams(use_tc_tiling_on_sc=False),
    )(t2, idxb)


def _kpconv_body(nx_ref, pq_ref, kpt_ref, w2_ref, res_ref, stats_ref):
    i = pl.program_id(0)
    q = pq_ref[:, 0:3]
    kpt = kpt_ref[...]
    wacc = [None] * KP
    for k in range(KNN):
        nxf = nx_ref[k, :, 0:DIM]
        sd = nx_ref[k, :, DIM:DIM + 3]
        diff = sd - q
        sq = None
        for d in range(3):
            term = diff[:, d:d + 1] - kpt[d:d + 1, :]
            sq = term * term if sq is None else sq + term * term
        wk = jnp.maximum(0.0, 1.0 - jnp.sqrt(sq) / KP_EXTENT)
        for p in range(KP):
            contrib = wk[:, p:p + 1] * nxf
            wacc[p] = contrib if wacc[p] is None else wacc[p] + contrib
    weighted = jnp.concatenate(wacc, axis=1)
    res = jnp.dot(weighted, w2_ref[...], preferred_element_type=jnp.float32)
    res_ref[...] = res

    @pl.when(i == 0)
    def _():
        stats_ref[...] = jnp.zeros_like(stats_ref)

    stats_ref[0:1, :] += jnp.sum(res, axis=0, keepdims=True)
    stats_ref[1:2, :] += jnp.sum(res * res, axis=0, keepdims=True)


def _kpconv_tc(nx, pq, kpt, w2, n_total, tile):
    nt = n_total // tile
    return pl.pallas_call(
        _kpconv_body,
        grid=(nt,),
        in_specs=[
            pl.BlockSpec((KNN, tile, FW), lambda i: (0, i, 0)),
            pl.BlockSpec((tile, 4), lambda i: (i, 0)),
            pl.BlockSpec((3, KP), lambda i: (0, 0)),
            pl.BlockSpec((KP * DIM, DIM), lambda i: (0, 0)),
        ],
        out_specs=[
            pl.BlockSpec((tile, DIM), lambda i: (i, 0)),
            pl.BlockSpec((2, DIM), lambda i: (0, 0)),
        ],
        out_shape=[
            jax.ShapeDtypeStruct((n_total, DIM), jnp.float32),
            jax.ShapeDtypeStruct((2, DIM), jnp.float32),
        ],
    )(nx, pq, kpt, w2)


def _head_body(inv_n, res_ref, stats_ref, g_ref, b_ref, hw_ref, hb_ref, out_ref):
    mean = stats_ref[0:1, :] * inv_n
    msq = stats_ref[1:2, :] * inv_n
    var = msq - mean * mean
    inv = lax.rsqrt(var + 1e-5)
    normed = (res_ref[...] - mean) * inv * g_ref[...] + b_ref[...]
    act = jnp.maximum(normed, 0.0)
    out_ref[...] = (
        jnp.dot(act, hw_ref[...], preferred_element_type=jnp.float32)
        + hb_ref[...]
    )


def _head_tc(res, stats, gamma, beta, hwt, hb, n_total, tile):
    nt = n_total // tile
    return pl.pallas_call(
        functools.partial(_head_body, 1.0 / n_total),
        grid=(nt,),
        in_specs=[
            pl.BlockSpec((tile, DIM), lambda i: (i, 0)),
            pl.BlockSpec((2, DIM), lambda i: (0, 0)),
            pl.BlockSpec((1, DIM), lambda i: (0, 0)),
            pl.BlockSpec((1, DIM), lambda i: (0, 0)),
            pl.BlockSpec((DIM, NUM_CLASSES), lambda i: (0, 0)),
            pl.BlockSpec((1, NUM_CLASSES), lambda i: (0, 0)),
        ],
        out_specs=pl.BlockSpec((tile, NUM_CLASSES), lambda i: (i, 0)),
        out_shape=jax.ShapeDtypeStruct((n_total, NUM_CLASSES), jnp.float32),
    )(res, stats, gamma, beta, hwt, hb)


def kernel(x, skip, px, py, pxyz, pknn, num_points, kernel_points, kp_weights,
           bn_gamma, bn_beta, head_w, head_b):
    del skip, num_points
    b = x.shape[0]
    hw = x.shape[1]
    n_total = px.shape[0]
    npts = n_total // b
    n_per_w = n_total // NW
    nchunks = n_per_w // CHUNK

    xt = x.reshape(b * hw, DIM)

    ix = ((px + 1.0) * GS_W - 1.0) * 0.5
    iy = ((py + 1.0) * GS_H - 1.0) * 0.5
    x0f = jnp.floor(ix)
    y0f = jnp.floor(iy)
    wx = ix - x0f
    wy = iy - y0f
    x0 = jnp.clip(x0f.astype(jnp.int32), 0, GS_W - 1)
    x1 = jnp.clip(x0f.astype(jnp.int32) + 1, 0, GS_W - 1)
    y0 = jnp.clip(y0f.astype(jnp.int32), 0, GS_H - 1)
    y1 = jnp.clip(y0f.astype(jnp.int32) + 1, 0, GS_H - 1)
    narange = jnp.arange(n_total, dtype=jnp.int32)
    boff = (narange // npts) * hw
    r00 = boff + y0 * GS_W + x0
    r01 = boff + y0 * GS_W + x1
    r10 = boff + y1 * GS_W + x0
    r11 = boff + y1 * GS_W + x1
    idxa = (jnp.stack([r00, r01, r10, r11], axis=1)
            .reshape(NW, nchunks, CHUNK, 4).transpose(0, 1, 3, 2))
    wa = (jnp.stack([(1 - wx) * (1 - wy), wx * (1 - wy),
                     (1 - wx) * wy, wx * wy], axis=1)
          .reshape(NW, nchunks, CHUNK, 4).transpose(0, 1, 3, 2)
          .reshape(NW, nchunks, 4 * CHUNK))
    pz16 = jnp.concatenate(
        [pxyz, jnp.zeros((n_total, 16 - 3), jnp.float32)], axis=1)

    t2 = _grid_sample_sc(xt, idxa, wa, pz16, n_total)

    gidx = pknn + (narange[:, None] // npts) * npts
    idxb = gidx.reshape(NW, nchunks, CHUNK, KNN).transpose(0, 3, 1, 2)
    nx = _neighbor_gather_sc(t2, idxb, n_total)

    pq = jnp.concatenate([pxyz, jnp.zeros((n_total, 1), jnp.float32)], axis=1)
    kpt = kernel_points.T
    w2 = kp_weights.reshape(KP * DIM, DIM)
    res, stats = _kpconv_tc(nx, pq, kpt, w2, n_total, tile=256)

    out = _head_tc(res, stats, bn_gamma.reshape(1, DIM), bn_beta.reshape(1, DIM),
                   head_w.T, head_b.reshape(1, NUM_CLASSES), n_total, tile=256)
    return out[:, :, None, None]


# speedup vs baseline: 6.9936x; 2.7397x over previous
import functools

import jax
import jax.numpy as jnp
from jax import lax
from jax.experimental import pallas as pl
from jax.experimental.pallas import tpu as pltpu
from jax.experimental.pallas import tpu_sc as plsc

DIM = 64
NUM_CLASSES = 17
GS_H, GS_W = 32, 128
KNN = 7
KP = 15
KP_EXTENT = 1.2

NC, NS = 2, 16
NW = NC * NS
CHUNK = 128
FW = 80


def _bcast_lane(vec, lane):
    idx = jnp.full((16, 1), lane, jnp.int32)
    return lax.gather(
        vec, idx,
        dimension_numbers=lax.GatherDimensionNumbers(
            offset_dims=(), collapsed_slice_dims=(0,), start_index_map=(0,)),
        slice_sizes=(1,),
        mode=lax.GatherScatterMode.PROMISE_IN_BOUNDS)


def _sc_mesh():
    return plsc.VectorSubcoreMesh(core_axis_name="c", subcore_axis_name="s")


def _wid():
    return lax.axis_index("s") * NC + lax.axis_index("c")


def _gs_body(n_per_w, nchunks, xt_hbm, idxa_hbm, wa_hbm, pz_hbm, out_hbm,
             idx_v, w_v, v_v, pz_v, fb, sem):
    wid = _wid()
    base = wid * n_per_w

    def chunk(c, carry):
        rows = base + c * CHUNK
        pltpu.sync_copy(idxa_hbm.at[wid, c], idx_v)
        pltpu.sync_copy(wa_hbm.at[wid, c], w_v)
        pltpu.sync_copy(pz_hbm.at[pl.ds(rows, CHUNK)], pz_v)
        descs = []
        for q in range(4):
            descs.append(pltpu.async_copy(xt_hbm.at[idx_v.at[q]], v_v.at[q], sem))
        for d in descs:
            d.wait()

        def blend(g, carry2):
            wvecs = [w_v[pl.ds(q * CHUNK + g * 16, 16)] for q in range(4)]
            for j2 in range(16):
                j = g * 16 + j2
                ws = [_bcast_lane(wvecs[q], j2) for q in range(4)]
                for cc in range(4):
                    sl = pl.ds(cc * 16, 16)
                    acc = ws[0] * v_v[0, j, sl]
                    acc = acc + ws[1] * v_v[1, j, sl]
                    acc = acc + ws[2] * v_v[2, j, sl]
                    acc = acc + ws[3] * v_v[3, j, sl]
                    fb[j, sl] = acc
                fb[j, pl.ds(64, 16)] = pz_v[j, :]
            return carry2

        lax.fori_loop(0, CHUNK // 16, blend, 0, unroll=False)
        pltpu.sync_copy(fb, out_hbm.at[pl.ds(rows, CHUNK)])
        return carry

    lax.fori_loop(0, nchunks, chunk, 0, unroll=False)


def _grid_sample_sc(xt, idxa, wa, pz16, n_total):
    n_per_w = n_total // NW
    nchunks = n_per_w // CHUNK
    body = functools.partial(_gs_body, n_per_w, nchunks)
    return pl.kernel(
        body,
        out_type=jax.ShapeDtypeStruct((n_total, FW), jnp.float32),
        mesh=_sc_mesh(),
        scratch_types=[
            pltpu.VMEM((4, CHUNK), jnp.int32),
            pltpu.VMEM((4 * CHUNK,), jnp.float32),
            pltpu.VMEM((4, CHUNK, DIM), jnp.float32),
            pltpu.VMEM((CHUNK, 16), jnp.float32),
            pltpu.VMEM((CHUNK, FW), jnp.float32),
            pltpu.SemaphoreType.DMA,
        ],
        compiler_params=pltpu.CompilerParams(use_tc_tiling_on_sc=False),
    )(xt, idxa, wa, pz16)


def _nb_body(n_per_w, nchunks, t2_hbm, idxb_hbm, out_hbm,
             idx_v, buf0, buf1, sem0, sem1):
    wid = _wid()
    base = wid * n_per_w
    pltpu.sync_copy(idxb_hbm.at[wid], idx_v)
    total = KNN * nchunks

    def issue(f, buf, sem):
        k = f // nchunks
        c = f - k * nchunks
        return pltpu.async_copy(t2_hbm.at[idx_v.at[k, c]], buf, sem)

    def wait(f, buf, sem):
        k = f // nchunks
        c = f - k * nchunks
        pltpu.make_async_copy(t2_hbm.at[idx_v.at[k, c]], buf, sem).wait()

    def store(f, buf):
        k = f // nchunks
        c = f - k * nchunks
        pltpu.sync_copy(buf, out_hbm.at[k, pl.ds(base + c * CHUNK, CHUNK)])

    issue(0, buf0, sem0)

    def pair(g, carry):
        f0 = 2 * g
        f1 = 2 * g + 1
        issue(f1, buf1, sem1)
        wait(f0, buf0, sem0)
        store(f0, buf0)

        @pl.when(f1 + 1 < total)
        def _():
            issue(f1 + 1, buf0, sem0)

        wait(f1, buf1, sem1)
        store(f1, buf1)
        return carry

    lax.fori_loop(0, total // 2, pair, 0, unroll=False)


def _neighbor_gather_sc(t2, idxb, n_total):
    n_per_w = n_total // NW
    nchunks = n_per_w // CHUNK
    body = functools.partial(_nb_body, n_per_w, nchunks)
    return pl.kernel(
        body,
        out_type=jax.ShapeDtypeStruct((KNN, n_total, FW), jnp.float32),
        mesh=_sc_mesh(),
        scratch_types=[
            pltpu.VMEM((KNN, nchunks, CHUNK), jnp.int32),
            pltpu.VMEM((CHUNK, FW), jnp.float32),
            pltpu.VMEM((CHUNK, FW), jnp.float32),
            pltpu.SemaphoreType.DMA,
            pltpu.SemaphoreType.DMA,
        ],
        compiler_params=pltpu.CompilerParams(use_tc_tiling_on_sc=False),
    )(t2, idxb)


def _kpconv_body(tile, nx_ref, qt_ref, kpt_ref, w2t_ref, res_ref, stats_ref):
    i = pl.program_id(0)
    kbc = [jnp.broadcast_to(kpt_ref[:, d:d + 1], (16, tile)) for d in range(3)]
    wacc = [None] * KP
    for k in range(KNN):
        nxt = nx_ref[k, :, 0:DIM + 3].T
        xkt = nxt[0:DIM, :]
        sq = None
        for d in range(3):
            diff = nxt[DIM + d:DIM + d + 1, :] - qt_ref[d:d + 1, :]
            term = jnp.broadcast_to(diff, (16, tile)) - kbc[d]
            sq = term * term if sq is None else sq + term * term
        wk = jnp.maximum(0.0, 1.0 - jnp.sqrt(sq) / KP_EXTENT)
        for p in range(KP):
            contrib = jnp.broadcast_to(wk[p:p + 1, :], (DIM, tile)) * xkt
            wacc[p] = contrib if wacc[p] is None else wacc[p] + contrib
    weighted = jnp.concatenate(wacc, axis=0)
    rest = jnp.dot(w2t_ref[...], weighted, preferred_element_type=jnp.float32)
    res = rest.T
    res_ref[...] = res

    @pl.when(i == 0)
    def _():
        stats_ref[...] = jnp.zeros_like(stats_ref)

    stats_ref[0:1, :] += jnp.sum(res, axis=0, keepdims=True)
    stats_ref[1:2, :] += jnp.sum(res * res, axis=0, keepdims=True)


def _kpconv_tc(nx, qt, kpt, w2t, n_total, tile):
    nt = n_total // tile
    return pl.pallas_call(
        functools.partial(_kpconv_body, tile),
        grid=(nt,),
        in_specs=[
            pl.BlockSpec((KNN, tile, FW), lambda i: (0, i, 0)),
            pl.BlockSpec((4, tile), lambda i: (0, i)),
            pl.BlockSpec((16, 4), lambda i: (0, 0)),
            pl.BlockSpec((DIM, KP * DIM), lambda i: (0, 0)),
        ],
        out_specs=[
            pl.BlockSpec((tile, DIM), lambda i: (i, 0)),
            pl.BlockSpec((2, DIM), lambda i: (0, 0)),
        ],
        out_shape=[
            jax.ShapeDtypeStruct((n_total, DIM), jnp.float32),
            jax.ShapeDtypeStruct((2, DIM), jnp.float32),
        ],
    )(nx, qt, kpt, w2t)


def _head_body(inv_n, res_ref, stats_ref, g_ref, b_ref, hw_ref, hb_ref, out_ref):
    mean = stats_ref[0:1, :] * inv_n
    msq = stats_ref[1:2, :] * inv_n
    var = msq - mean * mean
    inv = lax.rsqrt(var + 1e-5)
    normed = (res_ref[...] - mean) * inv * g_ref[...] + b_ref[...]
    act = jnp.maximum(normed, 0.0)
    out_ref[...] = (
        jnp.dot(act, hw_ref[...], preferred_element_type=jnp.float32)
        + hb_ref[...]
    )


def _head_tc(res, stats, gamma, beta, hwt, hb, n_total, tile):
    nt = n_total // tile
    return pl.pallas_call(
        functools.partial(_head_body, 1.0 / n_total),
        grid=(nt,),
        in_specs=[
            pl.BlockSpec((tile, DIM), lambda i: (i, 0)),
            pl.BlockSpec((2, DIM), lambda i: (0, 0)),
            pl.BlockSpec((1, DIM), lambda i: (0, 0)),
            pl.BlockSpec((1, DIM), lambda i: (0, 0)),
            pl.BlockSpec((DIM, NUM_CLASSES), lambda i: (0, 0)),
            pl.BlockSpec((1, NUM_CLASSES), lambda i: (0, 0)),
        ],
        out_specs=pl.BlockSpec((tile, NUM_CLASSES), lambda i: (i, 0)),
        out_shape=jax.ShapeDtypeStruct((n_total, NUM_CLASSES), jnp.float32),
    )(res, stats, gamma, beta, hwt, hb)


def kernel(x, skip, px, py, pxyz, pknn, num_points, kernel_points, kp_weights,
           bn_gamma, bn_beta, head_w, head_b):
    del skip, num_points
    b = x.shape[0]
    hw = x.shape[1]
    n_total = px.shape[0]
    npts = n_total // b
    n_per_w = n_total // NW
    nchunks = n_per_w // CHUNK

    xt = x.reshape(b * hw, DIM)

    ix = ((px + 1.0) * GS_W - 1.0) * 0.5
    iy = ((py + 1.0) * GS_H - 1.0) * 0.5
    x0f = jnp.floor(ix)
    y0f = jnp.floor(iy)
    wx = ix - x0f
    wy = iy - y0f
    x0 = jnp.clip(x0f.astype(jnp.int32), 0, GS_W - 1)
    x1 = jnp.clip(x0f.astype(jnp.int32) + 1, 0, GS_W - 1)
    y0 = jnp.clip(y0f.astype(jnp.int32), 0, GS_H - 1)
    y1 = jnp.clip(y0f.astype(jnp.int32) + 1, 0, GS_H - 1)
    narange = jnp.arange(n_total, dtype=jnp.int32)
    boff = (narange // npts) * hw
    r00 = boff + y0 * GS_W + x0
    r01 = boff + y0 * GS_W + x1
    r10 = boff + y1 * GS_W + x0
    r11 = boff + y1 * GS_W + x1
    idxa = (jnp.stack([r00, r01, r10, r11], axis=1)
            .reshape(NW, nchunks, CHUNK, 4).transpose(0, 1, 3, 2))
    wa = (jnp.stack([(1 - wx) * (1 - wy), wx * (1 - wy),
                     (1 - wx) * wy, wx * wy], axis=1)
          .reshape(NW, nchunks, CHUNK, 4).transpose(0, 1, 3, 2)
          .reshape(NW, nchunks, 4 * CHUNK))
    pz16 = jnp.concatenate(
        [pxyz, jnp.zeros((n_total, 16 - 3), jnp.float32)], axis=1)

    t2 = _grid_sample_sc(xt, idxa, wa, pz16, n_total)

    gidx = pknn + (narange[:, None] // npts) * npts
    idxb = gidx.reshape(NW, nchunks, CHUNK, KNN).transpose(0, 3, 1, 2)
    nx = _neighbor_gather_sc(t2, idxb, n_total)

    qt = jnp.concatenate(
        [pxyz.T, jnp.zeros((1, n_total), jnp.float32)], axis=0)
    kpt = jnp.concatenate(
        [kernel_points, jnp.zeros((16 - KP, 3), jnp.float32)], axis=0)
    kpt = jnp.concatenate([kpt, jnp.zeros((16, 1), jnp.float32)], axis=1)
    w2t = kp_weights.reshape(KP * DIM, DIM).T
    res, stats = _kpconv_tc(nx, qt, kpt, w2t, n_total, tile=256)

    out = _head_tc(res, stats, bn_gamma.reshape(1, DIM), bn_beta.reshape(1, DIM),
                   head_w.T, head_b.reshape(1, NUM_CLASSES), n_total, tile=256)
    return out[:, :, None, None]


# speedup vs baseline: 8.1027x; 1.1586x over previous
import functools

import jax
import jax.numpy as jnp
from jax import lax
from jax.experimental import pallas as pl
from jax.experimental.pallas import tpu as pltpu
from jax.experimental.pallas import tpu_sc as plsc

DIM = 64
NUM_CLASSES = 17
GS_H, GS_W = 32, 128
KNN = 7
KP = 15
KP_EXTENT = 1.2

NC, NS = 2, 16
NW = NC * NS
CHUNK = 128
FW = 80


def _bcast_lane(vec, lane):
    idx = jnp.full((16, 1), lane, jnp.int32)
    return lax.gather(
        vec, idx,
        dimension_numbers=lax.GatherDimensionNumbers(
            offset_dims=(), collapsed_slice_dims=(0,), start_index_map=(0,)),
        slice_sizes=(1,),
        mode=lax.GatherScatterMode.PROMISE_IN_BOUNDS)


def _sc_mesh():
    return plsc.VectorSubcoreMesh(core_axis_name="c", subcore_axis_name="s")


def _wid():
    return lax.axis_index("s") * NC + lax.axis_index("c")


def _gs_body(n_per_w, nchunks, npts, hw, xt_hbm, px_hbm, py_hbm, pz_hbm,
             out_hbm, px_v, py_v, pz_v, idx_v, w_v, v_v, fb,
             gsem0, gsem1, osem0, osem1):
    wid = _wid()
    base = wid * n_per_w
    pltpu.sync_copy(px_hbm.at[pl.ds(base, n_per_w)], px_v)
    pltpu.sync_copy(py_hbm.at[pl.ds(base, n_per_w)], py_v)
    pltpu.sync_copy(pz_hbm.at[pl.ds(base, n_per_w)], pz_v)
    gsems = (gsem0, gsem1)
    osems = (osem0, osem1)

    def comp(c, s):
        for g in range(CHUNK // 16):
            p0 = c * CHUNK + g * 16
            gx = px_v[pl.ds(p0, 16)]
            gy = py_v[pl.ds(p0, 16)]
            ix = (gx + 1.0) * (GS_W / 2) - 0.5
            iy = (gy + 1.0) * (GS_H / 2) - 0.5
            xtf = ix.astype(jnp.int32).astype(jnp.float32)
            ytf = iy.astype(jnp.int32).astype(jnp.float32)
            x0f = jnp.where(xtf > ix, xtf - 1.0, xtf)
            y0f = jnp.where(ytf > iy, ytf - 1.0, ytf)
            wx = ix - x0f
            wy = iy - y0f
            x0i = x0f.astype(jnp.int32)
            y0i = y0f.astype(jnp.int32)
            zero = jnp.zeros((16,), jnp.int32)
            x0 = jnp.minimum(jnp.maximum(x0i, zero), GS_W - 1)
            x1 = jnp.minimum(jnp.maximum(x0i + 1, zero), GS_W - 1)
            y0 = jnp.minimum(jnp.maximum(y0i, zero), GS_H - 1)
            y1 = jnp.minimum(jnp.maximum(y0i + 1, zero), GS_H - 1)
            pts = lax.iota(jnp.int32, 16) + (base + p0)
            boff = jnp.where(pts >= npts, hw, 0).astype(jnp.int32)
            r0 = boff + y0 * GS_W
            r1 = boff + y1 * GS_W
            sl = pl.ds(g * 16, 16)
            idx_v[s, 0, sl] = r0 + x0
            idx_v[s, 1, sl] = r0 + x1
            idx_v[s, 2, sl] = r1 + x0
            idx_v[s, 3, sl] = r1 + x1
            w_v[s, 0, sl] = (1.0 - wx) * (1.0 - wy)
            w_v[s, 1, sl] = wx * (1.0 - wy)
            w_v[s, 2, sl] = (1.0 - wx) * wy
            w_v[s, 3, sl] = wx * wy

    def gdesc(s, q):
        return pltpu.make_async_copy(
            xt_hbm.at[idx_v.at[s, q]], v_v.at[s, q], gsems[s])

    def issue_g(s):
        for q in range(4):
            pltpu.async_copy(xt_hbm.at[idx_v.at[s, q]], v_v.at[s, q], gsems[s])

    def wait_g(s):
        for q in range(4):
            gdesc(s, q).wait()

    def blend(c, s):
        def grp(g, carry):
            wvecs = [w_v[s, q, pl.ds(g * 16, 16)] for q in range(4)]
            for j2 in range(16):
                j = g * 16 + j2
                ws = [_bcast_lane(wvecs[q], j2) for q in range(4)]
                for cc in range(4):
                    sl = pl.ds(cc * 16, 16)
                    acc = ws[0] * v_v[s, 0, j, sl]
                    acc = acc + ws[1] * v_v[s, 1, j, sl]
                    acc = acc + ws[2] * v_v[s, 2, j, sl]
                    acc = acc + ws[3] * v_v[s, 3, j, sl]
                    fb[s, j, sl] = acc
                fb[s, j, pl.ds(64, 16)] = pz_v[c * CHUNK + j, :]
            return carry

        lax.fori_loop(0, CHUNK // 16, grp, 0, unroll=False)

    def odesc(c, s):
        return pltpu.make_async_copy(
            fb.at[s], out_hbm.at[pl.ds(base + c * CHUNK, CHUNK)], osems[s])

    comp(0, 0)
    issue_g(0)
    comp(1, 1)
    issue_g(1)

    def pair(g2, carry):
        for s in range(2):
            c = 2 * g2 + s
            wait_g(s)

            @pl.when(g2 > 0)
            def _():
                odesc(c, s).wait()

            blend(c, s)
            pltpu.async_copy(
                fb.at[s], out_hbm.at[pl.ds(base + c * CHUNK, CHUNK)], osems[s])

            @pl.when(c + 2 < nchunks)
            def _():
                comp(c + 2, s)
                issue_g(s)

        return carry

    lax.fori_loop(0, nchunks // 2, pair, 0, unroll=False)
    odesc(nchunks - 2, 0).wait()
    odesc(nchunks - 1, 1).wait()


def _grid_sample_sc(xt, px, py, pz16, n_total, npts, hw):
    n_per_w = n_total // NW
    nchunks = n_per_w // CHUNK
    body = functools.partial(_gs_body, n_per_w, nchunks, npts, hw)
    return pl.kernel(
        body,
        out_type=jax.ShapeDtypeStruct((n_total, FW), jnp.float32),
        mesh=_sc_mesh(),
        scratch_types=[
            pltpu.VMEM((n_per_w,), jnp.float32),
            pltpu.VMEM((n_per_w,), jnp.float32),
            pltpu.VMEM((n_per_w, 16), jnp.float32),
            pltpu.VMEM((2, 4, CHUNK), jnp.int32),
            pltpu.VMEM((2, 4, CHUNK), jnp.float32),
            pltpu.VMEM((2, 4, CHUNK, DIM), jnp.float32),
            pltpu.VMEM((2, CHUNK, FW), jnp.float32),
            pltpu.SemaphoreType.DMA,
            pltpu.SemaphoreType.DMA,
            pltpu.SemaphoreType.DMA,
            pltpu.SemaphoreType.DMA,
        ],
        compiler_params=pltpu.CompilerParams(use_tc_tiling_on_sc=False),
    )(xt, px, py, pz16)


_NBUF = 8


def _nb_body(n_per_w, nchunks, t2_hbm, idxb_hbm, out_hbm,
             idx_v, bufs, gsems, ssems):
    wid = _wid()
    base = wid * n_per_w
    pltpu.sync_copy(idxb_hbm.at[wid], idx_v)
    total = KNN * nchunks

    def gdesc(f, s):
        k = f // nchunks
        c = f - k * nchunks
        return pltpu.make_async_copy(
            t2_hbm.at[idx_v.at[k, c]], bufs.at[s], gsems.at[s])

    def sdesc(f, s):
        k = f // nchunks
        c = f - k * nchunks
        return pltpu.make_async_copy(
            bufs.at[s], out_hbm.at[k, pl.ds(base + c * CHUNK, CHUNK)],
            ssems.at[s])

    for s in range(_NBUF):
        gdesc(s, s).start()

    def ring(g, carry):
        for s in range(_NBUF):
            f = _NBUF * g + s
            gdesc(f, s).wait()
            sdesc(f, s).start()
        for s in range(_NBUF):
            f = _NBUF * g + s

            @pl.when(f + _NBUF < total)
            def _():
                sdesc(f, s).wait()
                gdesc(f + _NBUF, s).start()

        return carry

    lax.fori_loop(0, total // _NBUF, ring, 0, unroll=False)
    for s in range(_NBUF):
        sdesc(total - _NBUF + s, s).wait()


def _neighbor_gather_sc(t2, idxb, n_total):
    n_per_w = n_total // NW
    nchunks = n_per_w // CHUNK
    body = functools.partial(_nb_body, n_per_w, nchunks)
    return pl.kernel(
        body,
        out_type=jax.ShapeDtypeStruct((KNN, n_total, FW), jnp.float32),
        mesh=_sc_mesh(),
        scratch_types=[
            pltpu.VMEM((KNN, nchunks, CHUNK), jnp.int32),
            pltpu.VMEM((_NBUF, CHUNK, FW), jnp.float32),
            pltpu.SemaphoreType.DMA((_NBUF,)),
            pltpu.SemaphoreType.DMA((_NBUF,)),
        ],
        compiler_params=pltpu.CompilerParams(use_tc_tiling_on_sc=False),
    )(t2, idxb)


def _kpconv_body(tile, nx_ref, qt_ref, kpt_ref, w2t_ref, res_ref, stats_ref):
    i = pl.program_id(0)
    kbc = [jnp.broadcast_to(kpt_ref[:, d:d + 1], (16, tile)) for d in range(3)]
    wacc = [None] * KP
    for k in range(KNN):
        nxt = nx_ref[k, :, 0:DIM + 3].T
        xkt = nxt[0:DIM, :]
        sq = None
        for d in range(3):
            diff = nxt[DIM + d:DIM + d + 1, :] - qt_ref[d:d + 1, :]
            term = jnp.broadcast_to(diff, (16, tile)) - kbc[d]
            sq = term * term if sq is None else sq + term * term
        wk = jnp.maximum(0.0, 1.0 - jnp.sqrt(sq) / KP_EXTENT)
        for p in range(KP):
            contrib = jnp.broadcast_to(wk[p:p + 1, :], (DIM, tile)) * xkt
            wacc[p] = contrib if wacc[p] is None else wacc[p] + contrib
    weighted = jnp.concatenate(wacc, axis=0)
    rest = jnp.dot(w2t_ref[...], weighted, preferred_element_type=jnp.float32)
    res = rest.T
    res_ref[...] = res

    @pl.when(i == 0)
    def _():
        stats_ref[...] = jnp.zeros_like(stats_ref)

    stats_ref[0:1, :] += jnp.sum(res, axis=0, keepdims=True)
    stats_ref[1:2, :] += jnp.sum(res * res, axis=0, keepdims=True)


def _kpconv_tc(nx, qt, kpt, w2t, n_total, tile):
    nt = n_total // tile
    return pl.pallas_call(
        functools.partial(_kpconv_body, tile),
        grid=(nt,),
        in_specs=[
            pl.BlockSpec((KNN, tile, FW), lambda i: (0, i, 0)),
            pl.BlockSpec((4, tile), lambda i: (0, i)),
            pl.BlockSpec((16, 4), lambda i: (0, 0)),
            pl.BlockSpec((DIM, KP * DIM), lambda i: (0, 0)),
        ],
        out_specs=[
            pl.BlockSpec((tile, DIM), lambda i: (i, 0)),
            pl.BlockSpec((2, DIM), lambda i: (0, 0)),
        ],
        out_shape=[
            jax.ShapeDtypeStruct((n_total, DIM), jnp.float32),
            jax.ShapeDtypeStruct((2, DIM), jnp.float32),
        ],
    )(nx, qt, kpt, w2t)


def _head_body(inv_n, res_ref, stats_ref, g_ref, b_ref, hw_ref, hb_ref, out_ref):
    mean = stats_ref[0:1, :] * inv_n
    msq = stats_ref[1:2, :] * inv_n
    var = msq - mean * mean
    inv = lax.rsqrt(var + 1e-5)
    normed = (res_ref[...] - mean) * inv * g_ref[...] + b_ref[...]
    act = jnp.maximum(normed, 0.0)
    out_ref[...] = (
        jnp.dot(act, hw_ref[...], preferred_element_type=jnp.float32)
        + hb_ref[...]
    )


def _head_tc(res, stats, gamma, beta, hwt, hb, n_total, tile):
    nt = n_total // tile
    return pl.pallas_call(
        functools.partial(_head_body, 1.0 / n_total),
        grid=(nt,),
        in_specs=[
            pl.BlockSpec((tile, DIM), lambda i: (i, 0)),
            pl.BlockSpec((2, DIM), lambda i: (0, 0)),
            pl.BlockSpec((1, DIM), lambda i: (0, 0)),
            pl.BlockSpec((1, DIM), lambda i: (0, 0)),
            pl.BlockSpec((DIM, NUM_CLASSES), lambda i: (0, 0)),
            pl.BlockSpec((1, NUM_CLASSES), lambda i: (0, 0)),
        ],
        out_specs=pl.BlockSpec((tile, NUM_CLASSES), lambda i: (i, 0)),
        out_shape=jax.ShapeDtypeStruct((n_total, NUM_CLASSES), jnp.float32),
    )(res, stats, gamma, beta, hwt, hb)


def kernel(x, skip, px, py, pxyz, pknn, num_points, kernel_points, kp_weights,
           bn_gamma, bn_beta, head_w, head_b):
    del skip, num_points
    b = x.shape[0]
    hw = x.shape[1]
    n_total = px.shape[0]
    npts = n_total // b
    n_per_w = n_total // NW
    nchunks = n_per_w // CHUNK

    xt = x.reshape(b * hw, DIM)
    pz16 = jnp.concatenate(
        [pxyz, jnp.zeros((n_total, 16 - 3), jnp.float32)], axis=1)

    t2 = _grid_sample_sc(xt, px, py, pz16, n_total, npts, hw)

    narange = jnp.arange(n_total, dtype=jnp.int32)
    gidx = pknn + (narange[:, None] // npts) * npts
    idxb = gidx.reshape(NW, nchunks, CHUNK, KNN).transpose(0, 3, 1, 2)
    nx = _neighbor_gather_sc(t2, idxb, n_total)

    qt = jnp.concatenate(
        [pxyz.T, jnp.zeros((1, n_total), jnp.float32)], axis=0)
    kpt = jnp.concatenate(
        [kernel_points, jnp.zeros((16 - KP, 3), jnp.float32)], axis=0)
    kpt = jnp.concatenate([kpt, jnp.zeros((16, 1), jnp.float32)], axis=1)
    w2t = kp_weights.reshape(KP * DIM, DIM).T
    res, stats = _kpconv_tc(nx, qt, kpt, w2t, n_total, tile=512)

    out = _head_tc(res, stats, bn_gamma.reshape(1, DIM), bn_beta.reshape(1, DIM),
                   head_w.T, head_b.reshape(1, NUM_CLASSES), n_total, tile=256)
    return out[:, :, None, None]


# speedup vs baseline: 9.0015x; 1.1109x over previous
import functools

import jax
import jax.numpy as jnp
from jax import lax
from jax.experimental import pallas as pl
from jax.experimental.pallas import tpu as pltpu
from jax.experimental.pallas import tpu_sc as plsc

DIM = 64
NUM_CLASSES = 17
GS_H, GS_W = 32, 128
KNN = 7
KP = 15
KP_EXTENT = 1.2

NC, NS = 2, 16
NW = NC * NS
CHUNK = 128
FW = 80


def _bcast_lane(vec, lane):
    idx = jnp.full((16, 1), lane, jnp.int32)
    return lax.gather(
        vec, idx,
        dimension_numbers=lax.GatherDimensionNumbers(
            offset_dims=(), collapsed_slice_dims=(0,), start_index_map=(0,)),
        slice_sizes=(1,),
        mode=lax.GatherScatterMode.PROMISE_IN_BOUNDS)


def _sc_mesh():
    return plsc.VectorSubcoreMesh(core_axis_name="c", subcore_axis_name="s")


def _wid():
    return lax.axis_index("s") * NC + lax.axis_index("c")


def _gs_body(n_per_w, nchunks, npts, hw, xt_hbm, px_hbm, py_hbm, pz_hbm,
             out_hbm, px_v, py_v, pz_v, idx_v, w_v, v_v, fb,
             gsem0, gsem1, osem0, osem1):
    wid = _wid()
    base = wid * n_per_w
    pltpu.sync_copy(px_hbm.at[pl.ds(base, n_per_w)], px_v)
    pltpu.sync_copy(py_hbm.at[pl.ds(base, n_per_w)], py_v)
    pltpu.sync_copy(pz_hbm.at[pl.ds(base, n_per_w)], pz_v)
    gsems = (gsem0, gsem1)
    osems = (osem0, osem1)

    def comp(c, s):
        for g in range(CHUNK // 16):
            p0 = c * CHUNK + g * 16
            gx = px_v[pl.ds(p0, 16)]
            gy = py_v[pl.ds(p0, 16)]
            ix = (gx + 1.0) * (GS_W / 2) - 0.5
            iy = (gy + 1.0) * (GS_H / 2) - 0.5
            xtf = ix.astype(jnp.int32).astype(jnp.float32)
            ytf = iy.astype(jnp.int32).astype(jnp.float32)
            x0f = jnp.where(xtf > ix, xtf - 1.0, xtf)
            y0f = jnp.where(ytf > iy, ytf - 1.0, ytf)
            wx = ix - x0f
            wy = iy - y0f
            x0i = x0f.astype(jnp.int32)
            y0i = y0f.astype(jnp.int32)
            zero = jnp.zeros((16,), jnp.int32)
            x0 = jnp.minimum(jnp.maximum(x0i, zero), GS_W - 1)
            x1 = jnp.minimum(jnp.maximum(x0i + 1, zero), GS_W - 1)
            y0 = jnp.minimum(jnp.maximum(y0i, zero), GS_H - 1)
            y1 = jnp.minimum(jnp.maximum(y0i + 1, zero), GS_H - 1)
            pts = lax.iota(jnp.int32, 16) + (base + p0)
            boff = jnp.where(pts >= npts, hw, 0).astype(jnp.int32)
            r0 = boff + y0 * GS_W
            r1 = boff + y1 * GS_W
            sl = pl.ds(g * 16, 16)
            idx_v[s, 0, sl] = r0 + x0
            idx_v[s, 1, sl] = r0 + x1
            idx_v[s, 2, sl] = r1 + x0
            idx_v[s, 3, sl] = r1 + x1
            w_v[s, 0, sl] = (1.0 - wx) * (1.0 - wy)
            w_v[s, 1, sl] = wx * (1.0 - wy)
            w_v[s, 2, sl] = (1.0 - wx) * wy
            w_v[s, 3, sl] = wx * wy

    def gdesc(s, q):
        return pltpu.make_async_copy(
            xt_hbm.at[idx_v.at[s, q]], v_v.at[s, q], gsems[s])

    def issue_g(s):
        for q in range(4):
            pltpu.async_copy(xt_hbm.at[idx_v.at[s, q]], v_v.at[s, q], gsems[s])

    def wait_g(s):
        for q in range(4):
            gdesc(s, q).wait()

    def blend(c, s):
        def grp(g, carry):
            wvecs = [w_v[s, q, pl.ds(g * 16, 16)] for q in range(4)]
            for j2 in range(16):
                j = g * 16 + j2
                ws = [_bcast_lane(wvecs[q], j2) for q in range(4)]
                for cc in range(4):
                    sl = pl.ds(cc * 16, 16)
                    acc = ws[0] * v_v[s, 0, j, sl]
                    acc = acc + ws[1] * v_v[s, 1, j, sl]
                    acc = acc + ws[2] * v_v[s, 2, j, sl]
                    acc = acc + ws[3] * v_v[s, 3, j, sl]
                    fb[s, j, sl] = acc
                fb[s, j, pl.ds(64, 16)] = pz_v[c * CHUNK + j, :]
            return carry

        lax.fori_loop(0, CHUNK // 16, grp, 0, unroll=False)

    def odesc(c, s):
        return pltpu.make_async_copy(
            fb.at[s], out_hbm.at[pl.ds(base + c * CHUNK, CHUNK)], osems[s])

    comp(0, 0)
    issue_g(0)
    comp(1, 1)
    issue_g(1)

    def pair(g2, carry):
        for s in range(2):
            c = 2 * g2 + s
            wait_g(s)

            @pl.when(g2 > 0)
            def _():
                odesc(c, s).wait()

            blend(c, s)
            pltpu.async_copy(
                fb.at[s], out_hbm.at[pl.ds(base + c * CHUNK, CHUNK)], osems[s])

            @pl.when(c + 2 < nchunks)
            def _():
                comp(c + 2, s)
                issue_g(s)

        return carry

    lax.fori_loop(0, nchunks // 2, pair, 0, unroll=False)
    odesc(nchunks - 2, 0).wait()
    odesc(nchunks - 1, 1).wait()


def _grid_sample_sc(xt, px, py, pz16, n_total, npts, hw):
    n_per_w = n_total // NW
    nchunks = n_per_w // CHUNK
    body = functools.partial(_gs_body, n_per_w, nchunks, npts, hw)
    return pl.kernel(
        body,
        out_type=jax.ShapeDtypeStruct((n_total, FW), jnp.float32),
        mesh=_sc_mesh(),
        scratch_types=[
            pltpu.VMEM((n_per_w,), jnp.float32),
            pltpu.VMEM((n_per_w,), jnp.float32),
            pltpu.VMEM((n_per_w, 16), jnp.float32),
            pltpu.VMEM((2, 4, CHUNK), jnp.int32),
            pltpu.VMEM((2, 4, CHUNK), jnp.float32),
            pltpu.VMEM((2, 4, CHUNK, DIM), jnp.float32),
            pltpu.VMEM((2, CHUNK, FW), jnp.float32),
            pltpu.SemaphoreType.DMA,
            pltpu.SemaphoreType.DMA,
            pltpu.SemaphoreType.DMA,
            pltpu.SemaphoreType.DMA,
        ],
        compiler_params=pltpu.CompilerParams(use_tc_tiling_on_sc=False),
    )(xt, px, py, pz16)


_NBUF = 8


def _nb_body(n_per_w, nchunks, t2_hbm, idxb_hbm, out_hbm,
             idx_v, bufs, gsems, ssems):
    wid = _wid()
    base = wid * n_per_w
    pltpu.sync_copy(idxb_hbm.at[wid], idx_v)
    total = KNN * nchunks

    def gdesc(f, s):
        k = f // nchunks
        c = f - k * nchunks
        return pltpu.make_async_copy(
            t2_hbm.at[idx_v.at[k, c]], bufs.at[s], gsems.at[s])

    def sdesc(f, s):
        k = f // nchunks
        c = f - k * nchunks
        return pltpu.make_async_copy(
            bufs.at[s], out_hbm.at[k, pl.ds(base + c * CHUNK, CHUNK)],
            ssems.at[s])

    for s in range(_NBUF):
        gdesc(s, s).start()

    def ring(g, carry):
        for s in range(_NBUF):
            f = _NBUF * g + s
            gdesc(f, s).wait()
            sdesc(f, s).start()
        for s in range(_NBUF):
            f = _NBUF * g + s

            @pl.when(f + _NBUF < total)
            def _():
                sdesc(f, s).wait()
                gdesc(f + _NBUF, s).start()

        return carry

    lax.fori_loop(0, total // _NBUF, ring, 0, unroll=False)
    for s in range(_NBUF):
        sdesc(total - _NBUF + s, s).wait()


def _neighbor_gather_sc(t2, idxb, n_total):
    n_per_w = n_total // NW
    nchunks = n_per_w // CHUNK
    body = functools.partial(_nb_body, n_per_w, nchunks)
    return pl.kernel(
        body,
        out_type=jax.ShapeDtypeStruct((KNN, n_total, FW), jnp.float32),
        mesh=_sc_mesh(),
        scratch_types=[
            pltpu.VMEM((KNN, nchunks, CHUNK), jnp.int32),
            pltpu.VMEM((_NBUF, CHUNK, FW), jnp.float32),
            pltpu.SemaphoreType.DMA((_NBUF,)),
            pltpu.SemaphoreType.DMA((_NBUF,)),
        ],
        compiler_params=pltpu.CompilerParams(use_tc_tiling_on_sc=False),
    )(t2, idxb)


def _kpconv_body(tile, n_total, nx_ref, qt_ref, kpt_ref, w2t_ref, g_ref,
                 b_ref, hw_ref, hb_ref, out_ref, res_v, stats_v):
    p = pl.program_id(0)
    i = pl.program_id(1)

    @pl.when(p == 0)
    def _phase0():
        kbc = [jnp.broadcast_to(kpt_ref[:, d:d + 1], (16, tile))
               for d in range(3)]
        wacc = [None] * KP
        for k in range(KNN):
            nxt = nx_ref[k, :, 0:DIM + 3].T
            xkt = nxt[0:DIM, :]
            sq = None
            for d in range(3):
                diff = nxt[DIM + d:DIM + d + 1, :] - qt_ref[d:d + 1, :]
                term = jnp.broadcast_to(diff, (16, tile)) - kbc[d]
                sq = term * term if sq is None else sq + term * term
            wk = jnp.maximum(0.0, 1.0 - jnp.sqrt(sq) / KP_EXTENT)
            for kp in range(KP):
                contrib = jnp.broadcast_to(wk[kp:kp + 1, :], (DIM, tile)) * xkt
                wacc[kp] = contrib if wacc[kp] is None else wacc[kp] + contrib
        weighted = jnp.concatenate(wacc, axis=0)
        rest = jnp.dot(w2t_ref[...], weighted,
                       preferred_element_type=jnp.float32)
        res = rest.T
        res_v[pl.ds(i * tile, tile), :] = res

        @pl.when(i == 0)
        def _():
            stats_v[...] = jnp.zeros_like(stats_v)

        stats_v[0:1, :] += jnp.sum(res, axis=0, keepdims=True)
        stats_v[1:2, :] += jnp.sum(res * res, axis=0, keepdims=True)

    @pl.when(p == 1)
    def _phase1():
        inv_n = 1.0 / n_total
        mean = stats_v[0:1, :] * inv_n
        msq = stats_v[1:2, :] * inv_n
        var = msq - mean * mean
        inv = lax.rsqrt(var + 1e-5)
        res = res_v[pl.ds(i * tile, tile), :]
        normed = (res - mean) * inv * g_ref[...] + b_ref[...]
        act = jnp.maximum(normed, 0.0)
        out_ref[...] = (
            jnp.dot(act, hw_ref[...], preferred_element_type=jnp.float32)
            + hb_ref[...]
        )


def _kpconv_tc(nx, qt, kpt, w2t, gamma, beta, hwt, hb, n_total, tile):
    nt = n_total // tile
    return pl.pallas_call(
        functools.partial(_kpconv_body, tile, n_total),
        grid=(2, nt),
        in_specs=[
            pl.BlockSpec((KNN, tile, FW), lambda p, i: (0, i * (1 - p), 0)),
            pl.BlockSpec((4, tile), lambda p, i: (0, i * (1 - p))),
            pl.BlockSpec((16, 4), lambda p, i: (0, 0)),
            pl.BlockSpec((DIM, KP * DIM), lambda p, i: (0, 0)),
            pl.BlockSpec((1, DIM), lambda p, i: (0, 0)),
            pl.BlockSpec((1, DIM), lambda p, i: (0, 0)),
            pl.BlockSpec((DIM, NUM_CLASSES), lambda p, i: (0, 0)),
            pl.BlockSpec((1, NUM_CLASSES), lambda p, i: (0, 0)),
        ],
        out_specs=pl.BlockSpec((tile, NUM_CLASSES), lambda p, i: (i, 0)),
        out_shape=jax.ShapeDtypeStruct((n_total, NUM_CLASSES), jnp.float32),
        scratch_shapes=[
            pltpu.VMEM((n_total, DIM), jnp.float32),
            pltpu.VMEM((2, DIM), jnp.float32),
        ],
    )(nx, qt, kpt, w2t, gamma, beta, hwt, hb)


def kernel(x, skip, px, py, pxyz, pknn, num_points, kernel_points, kp_weights,
           bn_gamma, bn_beta, head_w, head_b):
    del skip, num_points
    b = x.shape[0]
    hw = x.shape[1]
    n_total = px.shape[0]
    npts = n_total // b
    n_per_w = n_total // NW
    nchunks = n_per_w // CHUNK

    xt = x.reshape(b * hw, DIM)
    pz16 = jnp.concatenate(
        [pxyz, jnp.zeros((n_total, 16 - 3), jnp.float32)], axis=1)

    t2 = _grid_sample_sc(xt, px, py, pz16, n_total, npts, hw)

    narange = jnp.arange(n_total, dtype=jnp.int32)
    gidx = pknn + (narange[:, None] // npts) * npts
    idxb = gidx.reshape(NW, nchunks, CHUNK, KNN).transpose(0, 3, 1, 2)
    nx = _neighbor_gather_sc(t2, idxb, n_total)

    qt = jnp.concatenate(
        [pxyz.T, jnp.zeros((1, n_total), jnp.float32)], axis=0)
    kpt = jnp.concatenate(
        [kernel_points, jnp.zeros((16 - KP, 3), jnp.float32)], axis=0)
    kpt = jnp.concatenate([kpt, jnp.zeros((16, 1), jnp.float32)], axis=1)
    w2t = kp_weights.reshape(KP * DIM, DIM).T
    out = _kpconv_tc(nx, qt, kpt, w2t, bn_gamma.reshape(1, DIM),
                     bn_beta.reshape(1, DIM), head_w.T,
                     head_b.reshape(1, NUM_CLASSES), n_total, tile=512)
    return out[:, :, None, None]
